# baseline (speedup 1.0000x reference)
import functools

import jax
import jax.numpy as jnp
from jax import lax
from jax.experimental import pallas as pl
from jax.experimental.pallas import tpu as pltpu
from jax.experimental.pallas import tpu_sc as plsc

_B = 16384
_D = 32
_K = 8192
_L = 4
_RB = 256
_NB = _B // _RB


def _tc_body(x_ref, cb_ref, cbh_ref, q_ref, r_ref, i_ref, e_ref):
    step = pl.program_id(0)
    resid = x_ref[...]
    quant = jnp.zeros_like(resid)
    lane8 = lax.broadcasted_iota(jnp.int32, (1, 8), 1)
    iota_k = lax.broadcasted_iota(jnp.int32, (_RB, _K), 1)
    err_vec = jnp.zeros((1, 8), jnp.float32)
    idx_rows = []
    for l in range(_L):
        cb = cb_ref[l]
        sim = lax.dot_general(
            resid.astype(jnp.bfloat16), cbh_ref[l], (((1,), (1,)), ((), ())),
            preferred_element_type=jnp.float32,
        )
        m = jnp.max(sim, axis=1, keepdims=True)
        idx = jnp.min(jnp.where(sim == m, iota_k, _K), axis=1).astype(jnp.int32)
        onehot = (iota_k == idx[:, None]).astype(jnp.float32)
        hard = lax.dot_general(
            onehot, cb, (((1,), (0,)), ((), ())),
            preferred_element_type=jnp.float32,
            precision=lax.Precision.HIGHEST,
        )
        resid = resid - hard
        quant = quant + hard
        r_ref[:, l, :] = resid
        idx_rows.append(idx[None, :])
        err_vec = err_vec + jnp.sum(resid * resid) * (lane8 == l).astype(jnp.float32)
    q_ref[...] = quant
    i_ref[...] = jnp.concatenate(idx_rows, axis=0)

    @pl.when(step == 0)
    def _():
        cb_all = cb_ref[...]
        reg = jnp.sum(cb_all * cb_all)
        e_ref[...] = err_vec + reg * (lane8 == _L).astype(jnp.float32)

    @pl.when(step != 0)
    def _():
        e_ref[...] = e_ref[...] + err_vec


_tc_call = pl.pallas_call(
    _tc_body,
    grid=(_NB,),
    in_specs=[
        pl.BlockSpec((_RB, _D), lambda i: (i, 0)),
        pl.BlockSpec((_L, _K, _D), lambda i: (0, 0, 0)),
        pl.BlockSpec((_L, _K, _D), lambda i: (0, 0, 0)),
    ],
    out_specs=[
        pl.BlockSpec((_RB, _D), lambda i: (i, 0)),
        pl.BlockSpec((_RB, _L, _D), lambda i: (i, 0, 0)),
        pl.BlockSpec((_L, _RB), lambda i: (0, i)),
        pl.BlockSpec((1, 8), lambda i: (0, 0)),
    ],
    out_shape=[
        jax.ShapeDtypeStruct((_B, _D), jnp.float32),
        jax.ShapeDtypeStruct((_B, _L, _D), jnp.float32),
        jax.ShapeDtypeStruct((_L, _B), jnp.int32),
        jax.ShapeDtypeStruct((1, 8), jnp.float32),
    ],
)


def _sc_usage_body(idx_hbm, usage_hbm, idx_v, usage_v):
    nc = 2
    wid = lax.axis_index("s") * nc + lax.axis_index("c")

    @pl.when(wid < _L)
    def _():
        pltpu.sync_copy(idx_hbm.at[wid], idx_v)

        def zero(j, _):
            usage_v[pl.ds(j * 16, 16)] = jnp.zeros((16,), jnp.float32)
            return 0

        lax.fori_loop(0, _K // 16, zero, 0)
        ones = jnp.full((16,), 1.0, jnp.float32)

        def scatter(j, _):
            iv = idx_v[pl.ds(j * 16, 16)]
            plsc.store_scatter(usage_v, [iv], ones)
            return 0

        lax.fori_loop(0, _B // 16, scatter, 0)
        pltpu.sync_copy(usage_v, usage_hbm.at[wid])


@functools.cache
def _sc_usage():
    return pl.kernel(
        _sc_usage_body,
        out_type=jax.ShapeDtypeStruct((_L, _K), jnp.float32),
        mesh=plsc.VectorSubcoreMesh(core_axis_name="c", subcore_axis_name="s"),
        scratch_types=[
            pltpu.VMEM((_B,), jnp.int32),
            pltpu.VMEM((_K,), jnp.float32),
        ],
        compiler_params=pltpu.CompilerParams(needs_layout_passes=False),
    )


def kernel(x, codebooks, temperature):
    del temperature
    quant, res_stack, idx_lb, err = _tc_call(
        x, codebooks, codebooks.astype(jnp.bfloat16))
    usage = _sc_usage()(idx_lb)
    sums = err[0]
    layer_err = jnp.sum(sums[:_L]) / (_B * _D)
    reg = sums[_L] / _K
    residual_loss = (layer_err + 0.01 * reg / _L) / _D
    return quant, residual_loss, lax.stop_gradient(res_stack), usage


# speedup vs baseline: 4.5436x; 4.5436x over previous
import functools

import jax
import jax.numpy as jnp
from jax import lax
from jax.experimental import pallas as pl
from jax.experimental.pallas import tpu as pltpu
from jax.experimental.pallas import tpu_sc as plsc

_B = 16384
_D = 32
_K = 8192
_L = 4
_RB = 256
_NB = _B // _RB
_NC = _D * 3 + 3


def _tc_body(x_ref, t_ref, w_ref, q_ref, r_ref, i_ref, e_ref):
    step = pl.program_id(0)
    recip = t_ref[0, 0]
    resid = x_ref[...]
    quant = jnp.zeros_like(resid)
    lane8 = lax.broadcasted_iota(jnp.int32, (1, 8), 1)
    err_vec = jnp.zeros((1, 8), jnp.float32)
    idx_rows = []
    for l in range(_L):
        w = w_ref[l]
        sim = lax.dot_general(
            resid.astype(jnp.bfloat16), w[:, 0:_D], (((1,), (1,)), ((), ())),
            preferred_element_type=jnp.float32,
        )
        scaled = sim * recip
        m = jnp.max(scaled, axis=1, keepdims=True)
        eq = (scaled == m).astype(jnp.bfloat16)
        o = lax.dot_general(
            eq, w, (((1,), (0,)), ((), ())),
            preferred_element_type=jnp.float32,
        )
        cnt = o[:, _D * 3]

        def _fast(o=o):
            hard = o[:, 0:_D] + (o[:, _D:2 * _D] + o[:, 2 * _D:3 * _D])
            idx = (o[:, _D * 3 + 1] * 64.0 + o[:, _D * 3 + 2]).astype(jnp.int32)
            return hard, idx

        def _slow(scaled=scaled, m=m, w=w):
            iota_k = lax.broadcasted_iota(jnp.int32, (_RB, _K), 1)
            idx = jnp.min(jnp.where(scaled == m, iota_k, _K - 1), axis=1)
            idx = idx.astype(jnp.int32)
            oh = (iota_k == idx[:, None]).astype(jnp.bfloat16)
            o2 = lax.dot_general(
                oh, w[:, 0:3 * _D], (((1,), (0,)), ((), ())),
                preferred_element_type=jnp.float32,
            )
            hard = o2[:, 0:_D] + (o2[:, _D:2 * _D] + o2[:, 2 * _D:3 * _D])
            return hard, idx

        tie = jnp.any(cnt != 1.0)
        hard, idx = lax.cond(tie, _slow, _fast)
        resid = resid - hard
        quant = quant + hard
        r_ref[:, l, :] = resid
        idx_rows.append(idx[None, :])
        err_vec = err_vec + jnp.sum(resid * resid) * (lane8 == l).astype(jnp.float32)
    q_ref[...] = quant
    i_ref[...] = jnp.concatenate(idx_rows, axis=0)

    @pl.when(step == 0)
    def _():
        cb_all = w_ref[:, :, 0:_D].astype(jnp.float32) + (
            w_ref[:, :, _D:2 * _D].astype(jnp.float32)
            + w_ref[:, :, 2 * _D:3 * _D].astype(jnp.float32))
        reg = jnp.sum(cb_all * cb_all)
        e_ref[...] = err_vec + reg * (lane8 == _L).astype(jnp.float32)

    @pl.when(step != 0)
    def _():
        e_ref[...] = e_ref[...] + err_vec


_tc_call = pl.pallas_call(
    _tc_body,
    grid=(_NB,),
    in_specs=[
        pl.BlockSpec((_RB, _D), lambda i: (i, 0)),
        pl.BlockSpec((1, 1), lambda i: (0, 0)),
        pl.BlockSpec((_L, _K, _NC), lambda i: (0, 0, 0)),
    ],
    out_specs=[
        pl.BlockSpec((_RB, _D), lambda i: (i, 0)),
        pl.BlockSpec((_RB, _L, _D), lambda i: (i, 0, 0)),
        pl.BlockSpec((_L, _RB), lambda i: (0, i)),
        pl.BlockSpec((1, 8), lambda i: (0, 0)),
    ],
    out_shape=[
        jax.ShapeDtypeStruct((_B, _D), jnp.float32),
        jax.ShapeDtypeStruct((_B, _L, _D), jnp.float32),
        jax.ShapeDtypeStruct((_L, _B), jnp.int32),
        jax.ShapeDtypeStruct((1, 8), jnp.float32),
    ],
)


def _sc_usage_body(idx_hbm, usage_hbm, idx_v, usage_v):
    nc = 2
    wid = lax.axis_index("s") * nc + lax.axis_index("c")

    @pl.when(wid < _L)
    def _():
        pltpu.sync_copy(idx_hbm.at[wid], idx_v)

        def zero(j, _):
            usage_v[pl.ds(j * 16, 16)] = jnp.zeros((16,), jnp.float32)
            return 0

        lax.fori_loop(0, _K // 16, zero, 0)
        ones = jnp.full((16,), 1.0, jnp.float32)

        def scatter(j, _):
            iv = idx_v[pl.ds(j * 16, 16)]
            plsc.store_scatter(usage_v, [iv], ones)
            return 0

        lax.fori_loop(0, _B // 16, scatter, 0)
        pltpu.sync_copy(usage_v, usage_hbm.at[wid])


@functools.cache
def _sc_usage():
    return pl.kernel(
        _sc_usage_body,
        out_type=jax.ShapeDtypeStruct((_L, _K), jnp.float32),
        mesh=plsc.VectorSubcoreMesh(core_axis_name="c", subcore_axis_name="s"),
        scratch_types=[
            pltpu.VMEM((_B,), jnp.int32),
            pltpu.VMEM((_K,), jnp.float32),
        ],
        compiler_params=pltpu.CompilerParams(needs_layout_passes=False),
    )


def _build_w(codebooks):
    hi = codebooks.astype(jnp.bfloat16)
    hi_f32 = lax.bitcast_convert_type(
        lax.bitcast_convert_type(hi, jnp.uint16).astype(jnp.uint32) << 16,
        jnp.float32)
    r1 = codebooks - hi_f32
    r1b = lax.bitcast_convert_type(r1, jnp.uint32)
    mid = lax.bitcast_convert_type((r1b >> 16).astype(jnp.uint16), jnp.bfloat16)
    mid_f32 = lax.bitcast_convert_type(r1b & jnp.uint32(0xFFFF0000), jnp.float32)
    lo = (r1 - mid_f32).astype(jnp.bfloat16)
    kidx = jnp.arange(_K, dtype=jnp.int32)
    ones = jnp.ones((_L, _K, 1), jnp.bfloat16)
    ihi = jnp.broadcast_to((kidx >> 6).astype(jnp.bfloat16)[None, :, None], (_L, _K, 1))
    ilo = jnp.broadcast_to((kidx & 63).astype(jnp.bfloat16)[None, :, None], (_L, _K, 1))
    return jnp.concatenate([hi, mid, lo, ones, ihi, ilo], axis=2)


def kernel(x, codebooks, temperature):
    w = _build_w(codebooks)
    recip = 1.0 / jnp.maximum(temperature.astype(jnp.float32), 0.04)
    quant, res_stack, idx_lb, err = _tc_call(x, jnp.reshape(recip, (1, 1)), w)
    usage = _sc_usage()(idx_lb)
    sums = err[0]
    layer_err = jnp.sum(sums[:_L]) / (_B * _D)
    reg = sums[_L] / _K
    residual_loss = (layer_err + 0.01 * reg / _L) / _D
    return quant, residual_loss, lax.stop_gradient(res_stack), usage
